# Initial kernel scaffold; baseline (speedup 1.0000x reference)
import jax
import jax.numpy as jnp
from jax.experimental import pallas as pl


def kernel(probs, labels):
    raise NotImplementedError("write your pallas kernel here")


# speedup vs baseline: 1.7150x; 1.7150x over previous
import functools

import numpy as np
import jax
import jax.numpy as jnp
from jax import lax
from jax.experimental import pallas as pl
from jax.experimental.pallas import tpu as pltpu
from jax.experimental.pallas import tpu_sc as plsc

_N = 1048576
_C = 128
_NBINS = 15
_R = 512
_G = _N // _R
_NC = 2
_NS = 16
_NW = _NC * _NS
_CH = _N // _NW
_L = 16

_BOUNDS = [float(b) for b in np.linspace(0.0, 1.0, _NBINS + 1).astype(np.float32)]


def _tc_body(probs_ref, labels_ref, conf_ref, corr_ref):
    x = probs_ref[...]
    m = jnp.max(x, axis=1, keepdims=True)
    iota = lax.broadcasted_iota(jnp.int32, (_R, _C), 1)
    pred = jnp.min(jnp.where(x == m, iota, _C), axis=1)
    lab = labels_ref[0, 0, :]
    conf_ref[0, 0, :] = m[:, 0]
    corr_ref[0, 0, :] = (pred == lab).astype(jnp.float32)


_tc_call = pl.pallas_call(
    _tc_body,
    grid=(_G,),
    in_specs=[
        pl.BlockSpec((_R, _C), lambda i: (i, 0)),
        pl.BlockSpec((1, 1, _R), lambda i: (i, 0, 0)),
    ],
    out_specs=[
        pl.BlockSpec((1, 1, _R), lambda i: (i, 0, 0)),
        pl.BlockSpec((1, 1, _R), lambda i: (i, 0, 0)),
    ],
    out_shape=[
        jax.ShapeDtypeStruct((_G, 1, _R), jnp.float32),
        jax.ShapeDtypeStruct((_G, 1, _R), jnp.float32),
    ],
    compiler_params=pltpu.CompilerParams(
        dimension_semantics=("arbitrary",),
    ),
)


_PACK = float(2.0 ** -12)


def _sc_hist_body(conf_hbm, corr_hbm, out_hbm, conf_v, corr_v, pk_h, cs_h):
    wid = lax.axis_index("s") * _NC + lax.axis_index("c")
    base = wid * _CH
    pltpu.sync_copy(conf_hbm.at[pl.ds(base, _CH)], conf_v)
    pltpu.sync_copy(corr_hbm.at[pl.ds(base, _CH)], corr_v)

    zeros16 = jnp.zeros((_L,), jnp.float32)
    nrows = _NBINS - 1

    def body(i, carry):
        pks, css, cs_tot, as_tot = carry
        off = i * _L
        c = conf_v[pl.ds(off, _L)]
        a = corr_v[pl.ds(off, _L)]
        t = a * _PACK + 1.0
        new_pks = []
        new_css = []
        for j in range(1, _NBINS):
            m = c > _BOUNDS[j]
            new_pks.append(pks[j - 1] + jnp.where(m, t, 0.0))
            new_css.append(css[j - 1] + jnp.where(m, c, 0.0))
        return (tuple(new_pks), tuple(new_css), cs_tot + c, as_tot + a)

    init = (tuple(zeros16 for _ in range(nrows)),
            tuple(zeros16 for _ in range(nrows)), zeros16, zeros16)
    pks, css, cs_tot, as_tot = lax.fori_loop(0, _CH // _L, body, init)

    pk_h[0, :] = as_tot
    cs_h[0, :] = cs_tot
    for j in range(1, _NBINS):
        pk_h[j, :] = pks[j - 1]
        cs_h[j, :] = css[j - 1]
    pk_h[_NBINS, :] = zeros16
    cs_h[_NBINS, :] = zeros16

    pltpu.sync_copy(pk_h, out_hbm.at[wid, 0])
    pltpu.sync_copy(cs_h, out_hbm.at[wid, 1])


@functools.lru_cache(maxsize=1)
def _make_sc_hist():
    return pl.kernel(
        _sc_hist_body,
        mesh=plsc.VectorSubcoreMesh(core_axis_name="c", subcore_axis_name="s"),
        out_type=jax.ShapeDtypeStruct((_NW, 2, _L, _L), jnp.float32),
        scratch_types=[
            pltpu.VMEM((_CH,), jnp.float32),
            pltpu.VMEM((_CH,), jnp.float32),
            pltpu.VMEM((_L, _L), jnp.float32),
            pltpu.VMEM((_L, _L), jnp.float32),
        ],
    )


def kernel(probs, labels):
    labels3 = labels.reshape(_G, 1, _R)
    conf, corr = _tc_call(probs, labels3)
    parts = _make_sc_hist()(conf.reshape(_N), corr.reshape(_N))
    pk = parts[:, 0]
    cs = parts[:, 1]
    cnt_rows = jnp.floor(pk)
    acc_rows = (pk - cnt_rows) * 4096.0
    cnt_cum = jnp.sum(cnt_rows, axis=(0, 2))
    acc_cum = jnp.sum(acc_rows, axis=(0, 2))
    cs_cum = jnp.sum(cs, axis=(0, 2))
    cnt_cum = cnt_cum.at[0].set(float(_N))
    acc_cum = acc_cum.at[0].set(jnp.sum(parts[:, 0, 0, :]))
    cs_cum = cs_cum.at[0].set(jnp.sum(parts[:, 1, 0, :]))
    counts = cnt_cum[:_NBINS] - cnt_cum[1:_NBINS + 1]
    conf_sum = cs_cum[:_NBINS] - cs_cum[1:_NBINS + 1]
    acc_sum = acc_cum[:_NBINS] - acc_cum[1:_NBINS + 1]
    denom = jnp.maximum(counts, 1.0)
    per_bin = jnp.where(counts > 0,
                        jnp.abs(conf_sum / denom - acc_sum / denom)
                        * (counts / _N), 0.0)
    return jnp.sum(per_bin).reshape(1)


# speedup vs baseline: 2.3396x; 1.3642x over previous
import functools

import numpy as np
import jax
import jax.numpy as jnp
from jax import lax
from jax.experimental import pallas as pl
from jax.experimental.pallas import tpu as pltpu
from jax.experimental.pallas import tpu_sc as plsc

_N = 1048576
_C = 128
_NBINS = 15
_R2 = 1024
_G2 = _N // _R2
_NC = 2
_NS = 16
_NW = _NC * _NS
_CH = _N // _NW
_L = 16

_BOUNDS = [float(b) for b in np.linspace(0.0, 1.0, _NBINS + 1).astype(np.float32)]


def _tc_body(probs_ref, labels_ref, conf_ref, corr_ref):
    x = probs_ref[0]
    lab = labels_ref[0]
    iota = lax.broadcasted_iota(jnp.int32, (8, _C, _C), 2)
    xi = lax.bitcast_convert_type(x, jnp.int32)
    inv = 127 - iota
    mor = jnp.where(iota == lab, inv | 128, inv)
    key = (xi & -256) | mor
    final = jnp.max(key, axis=2)

    corr_ref[0] = jnp.where((final & 128) != 0, 1.0, 0.0)
    conf_ref[0] = lax.bitcast_convert_type((final & -256) | 64, jnp.float32)


_tc_call = pl.pallas_call(
    _tc_body,
    grid=(_G2,),
    in_specs=[
        pl.BlockSpec((1, 8, _C, _C), lambda i: (i, 0, 0, 0)),
        pl.BlockSpec((1, 8, _C, 1), lambda i: (i, 0, 0, 0)),
    ],
    out_specs=[
        pl.BlockSpec((1, 8, _C), lambda i: (i, 0, 0)),
        pl.BlockSpec((1, 8, _C), lambda i: (i, 0, 0)),
    ],
    out_shape=[
        jax.ShapeDtypeStruct((_G2, 8, _C), jnp.float32),
        jax.ShapeDtypeStruct((_G2, 8, _C), jnp.float32),
    ],
    compiler_params=pltpu.CompilerParams(
        dimension_semantics=("arbitrary",),
    ),
)


_PACK = float(2.0 ** -12)


def _sc_hist_body(conf_hbm, corr_hbm, out_hbm, conf_v, corr_v, pk_h, cs_h):
    wid = lax.axis_index("s") * _NC + lax.axis_index("c")
    base = wid * _CH
    pltpu.sync_copy(conf_hbm.at[pl.ds(base, _CH)], conf_v)
    pltpu.sync_copy(corr_hbm.at[pl.ds(base, _CH)], corr_v)

    zeros16 = jnp.zeros((_L,), jnp.float32)
    nrows = _NBINS - 1

    def body(i, carry):
        pks, css, cs_tot, as_tot = carry
        off = i * _L
        c = conf_v[pl.ds(off, _L)]
        a = corr_v[pl.ds(off, _L)]
        t = a * _PACK + 1.0
        new_pks = []
        new_css = []
        for j in range(1, _NBINS):
            m = c > _BOUNDS[j]
            new_pks.append(pks[j - 1] + jnp.where(m, t, 0.0))
            new_css.append(css[j - 1] + jnp.where(m, c, 0.0))
        return (tuple(new_pks), tuple(new_css), cs_tot + c, as_tot + a)

    init = (tuple(zeros16 for _ in range(nrows)),
            tuple(zeros16 for _ in range(nrows)), zeros16, zeros16)
    pks, css, cs_tot, as_tot = lax.fori_loop(0, _CH // _L, body, init)

    pk_h[0, :] = as_tot
    cs_h[0, :] = cs_tot
    for j in range(1, _NBINS):
        pk_h[j, :] = pks[j - 1]
        cs_h[j, :] = css[j - 1]
    pk_h[_NBINS, :] = zeros16
    cs_h[_NBINS, :] = zeros16

    pltpu.sync_copy(pk_h, out_hbm.at[wid, 0])
    pltpu.sync_copy(cs_h, out_hbm.at[wid, 1])


@functools.lru_cache(maxsize=1)
def _make_sc_hist():
    return pl.kernel(
        _sc_hist_body,
        mesh=plsc.VectorSubcoreMesh(core_axis_name="c", subcore_axis_name="s"),
        out_type=jax.ShapeDtypeStruct((_NW, 2, _L, _L), jnp.float32),
        scratch_types=[
            pltpu.VMEM((_CH,), jnp.float32),
            pltpu.VMEM((_CH,), jnp.float32),
            pltpu.VMEM((_L, _L), jnp.float32),
            pltpu.VMEM((_L, _L), jnp.float32),
        ],
    )


def kernel(probs, labels):
    probs4 = probs.reshape(_G2, 8, _C, _C)
    labels4 = labels.reshape(_G2, 8, _C, 1)
    conf, corr = _tc_call(probs4, labels4)
    parts = _make_sc_hist()(conf.reshape(_N), corr.reshape(_N))
    pk = parts[:, 0]
    cs = parts[:, 1]
    cnt_rows = jnp.floor(pk)
    acc_rows = (pk - cnt_rows) * 4096.0
    cnt_cum = jnp.sum(cnt_rows, axis=(0, 2))
    acc_cum = jnp.sum(acc_rows, axis=(0, 2))
    cs_cum = jnp.sum(cs, axis=(0, 2))
    cnt_cum = cnt_cum.at[0].set(float(_N))
    acc_cum = acc_cum.at[0].set(jnp.sum(parts[:, 0, 0, :]))
    cs_cum = cs_cum.at[0].set(jnp.sum(parts[:, 1, 0, :]))
    counts = cnt_cum[:_NBINS] - cnt_cum[1:_NBINS + 1]
    conf_sum = cs_cum[:_NBINS] - cs_cum[1:_NBINS + 1]
    acc_sum = acc_cum[:_NBINS] - acc_cum[1:_NBINS + 1]
    denom = jnp.maximum(counts, 1.0)
    per_bin = jnp.where(counts > 0,
                        jnp.abs(conf_sum / denom - acc_sum / denom)
                        * (counts / _N), 0.0)
    return jnp.sum(per_bin).reshape(1)


# speedup vs baseline: 3.2328x; 1.3817x over previous
import functools

import numpy as np
import jax
import jax.numpy as jnp
from jax import lax
from jax.experimental import pallas as pl
from jax.experimental.pallas import tpu as pltpu
from jax.experimental.pallas import tpu_sc as plsc

_N = 1048576
_C = 128
_NBINS = 15
_R2 = 1024
_G2 = _N // _R2
_NC = 2
_NS = 16
_NW = _NC * _NS
_CH = _N // _NW
_L = 16

_BOUNDS = [float(b) for b in np.linspace(0.0, 1.0, _NBINS + 1).astype(np.float32)]


def _tc_body(probs_ref, labels_ref, conf_ref, corr_ref):
    x = probs_ref[0]
    lab = labels_ref[0]
    iota = lax.broadcasted_iota(jnp.int32, (8, _C, _C), 2)
    xi = lax.bitcast_convert_type(x, jnp.int32)
    key = (xi & -128) | (127 - iota)
    final = jnp.max(key, axis=2)

    pred = 127 - (final & 127)
    corr_ref[0] = jnp.where(pred == lab, 1.0, 0.0)
    conf_ref[0] = lax.bitcast_convert_type((final & -128) | 64, jnp.float32)


_tc_call = pl.pallas_call(
    _tc_body,
    grid=(_G2,),
    in_specs=[
        pl.BlockSpec((1, 8, _C, _C), lambda i: (i, 0, 0, 0)),
        pl.BlockSpec((1, 8, _C), lambda i: (i, 0, 0)),
    ],
    out_specs=[
        pl.BlockSpec((1, 8, _C), lambda i: (i, 0, 0)),
        pl.BlockSpec((1, 8, _C), lambda i: (i, 0, 0)),
    ],
    out_shape=[
        jax.ShapeDtypeStruct((_G2, 8, _C), jnp.float32),
        jax.ShapeDtypeStruct((_G2, 8, _C), jnp.float32),
    ],
    compiler_params=pltpu.CompilerParams(
        dimension_semantics=("arbitrary",),
    ),
)


_PACK = float(2.0 ** -12)


def _sc_hist_body(conf_hbm, corr_hbm, out_hbm, conf_v, corr_v, pk_h, cs_h):
    wid = lax.axis_index("s") * _NC + lax.axis_index("c")
    base = wid * _CH
    pltpu.sync_copy(conf_hbm.at[pl.ds(base, _CH)], conf_v)
    pltpu.sync_copy(corr_hbm.at[pl.ds(base, _CH)], corr_v)

    zeros16 = jnp.zeros((_L,), jnp.float32)
    nrows = _NBINS - 1

    def body(i, carry):
        pks, css, cs_tot, as_tot = carry
        off = i * _L
        c = conf_v[pl.ds(off, _L)]
        a = corr_v[pl.ds(off, _L)]
        t = a * _PACK + 1.0
        new_pks = []
        new_css = []
        for j in range(1, _NBINS):
            m = c > _BOUNDS[j]
            new_pks.append(pks[j - 1] + jnp.where(m, t, 0.0))
            new_css.append(css[j - 1] + jnp.where(m, c, 0.0))
        return (tuple(new_pks), tuple(new_css), cs_tot + c, as_tot + a)

    init = (tuple(zeros16 for _ in range(nrows)),
            tuple(zeros16 for _ in range(nrows)), zeros16, zeros16)
    pks, css, cs_tot, as_tot = lax.fori_loop(0, _CH // _L, body, init)

    pk_h[0, :] = as_tot
    cs_h[0, :] = cs_tot
    for j in range(1, _NBINS):
        pk_h[j, :] = pks[j - 1]
        cs_h[j, :] = css[j - 1]
    pk_h[_NBINS, :] = zeros16
    cs_h[_NBINS, :] = zeros16

    pltpu.sync_copy(pk_h, out_hbm.at[wid, 0])
    pltpu.sync_copy(cs_h, out_hbm.at[wid, 1])


@functools.lru_cache(maxsize=1)
def _make_sc_hist():
    return pl.kernel(
        _sc_hist_body,
        mesh=plsc.VectorSubcoreMesh(core_axis_name="c", subcore_axis_name="s"),
        out_type=jax.ShapeDtypeStruct((_NW, 2, _L, _L), jnp.float32),
        scratch_types=[
            pltpu.VMEM((_CH,), jnp.float32),
            pltpu.VMEM((_CH,), jnp.float32),
            pltpu.VMEM((_L, _L), jnp.float32),
            pltpu.VMEM((_L, _L), jnp.float32),
        ],
    )


def kernel(probs, labels):
    probs4 = probs.reshape(_G2, 8, _C, _C)
    labels4 = labels.reshape(_G2, 8, _C)
    conf, corr = _tc_call(probs4, labels4)
    parts = _make_sc_hist()(conf.reshape(_N), corr.reshape(_N))
    pk = parts[:, 0]
    cs = parts[:, 1]
    cnt_rows = jnp.floor(pk)
    acc_rows = (pk - cnt_rows) * 4096.0
    cnt_cum = jnp.sum(cnt_rows, axis=(0, 2))
    acc_cum = jnp.sum(acc_rows, axis=(0, 2))
    cs_cum = jnp.sum(cs, axis=(0, 2))
    cnt_cum = cnt_cum.at[0].set(float(_N))
    acc_cum = acc_cum.at[0].set(jnp.sum(parts[:, 0, 0, :]))
    cs_cum = cs_cum.at[0].set(jnp.sum(parts[:, 1, 0, :]))
    counts = cnt_cum[:_NBINS] - cnt_cum[1:_NBINS + 1]
    conf_sum = cs_cum[:_NBINS] - cs_cum[1:_NBINS + 1]
    acc_sum = acc_cum[:_NBINS] - acc_cum[1:_NBINS + 1]
    denom = jnp.maximum(counts, 1.0)
    per_bin = jnp.where(counts > 0,
                        jnp.abs(conf_sum / denom - acc_sum / denom)
                        * (counts / _N), 0.0)
    return jnp.sum(per_bin).reshape(1)


# speedup vs baseline: 4.5237x; 1.3993x over previous
import functools

import numpy as np
import jax
import jax.numpy as jnp
from jax import lax
from jax.experimental import pallas as pl
from jax.experimental.pallas import tpu as pltpu
from jax.experimental.pallas import tpu_sc as plsc

_N = 1048576
_C = 128
_NBINS = 15
_R2 = 2048
_G2 = _N // _R2
_SB = _R2 // _C
_NC = 2
_NS = 16
_NW = _NC * _NS
_CH = _N // _NW
_L = 16

_BOUNDS = [float(b) for b in np.linspace(0.0, 1.0, _NBINS + 1).astype(np.float32)]


def _tc_body(probs_ref, labels_ref, conf_ref, corr_ref):
    x = probs_ref[0]
    lab = labels_ref[0]
    iota = lax.broadcasted_iota(jnp.int32, (_SB, _C, _C), 2)
    xi = lax.bitcast_convert_type(x, jnp.int32)
    key = (xi & -128) | (127 - iota)
    final = jnp.max(key, axis=2)

    pred = 127 - (final & 127)
    corr_ref[0] = jnp.where(pred == lab, 1.0, 0.0)
    conf_ref[0] = lax.bitcast_convert_type((final & -128) | 64, jnp.float32)


_tc_call = pl.pallas_call(
    _tc_body,
    grid=(_G2,),
    in_specs=[
        pl.BlockSpec((1, _SB, _C, _C), lambda i: (i, 0, 0, 0)),
        pl.BlockSpec((1, _SB, _C), lambda i: (i, 0, 0)),
    ],
    out_specs=[
        pl.BlockSpec((1, _SB, _C), lambda i: (i, 0, 0)),
        pl.BlockSpec((1, _SB, _C), lambda i: (i, 0, 0)),
    ],
    out_shape=[
        jax.ShapeDtypeStruct((_G2, _SB, _C), jnp.float32),
        jax.ShapeDtypeStruct((_G2, _SB, _C), jnp.float32),
    ],
    compiler_params=pltpu.CompilerParams(
        dimension_semantics=("arbitrary",),
    ),
)


_PACK = float(2.0 ** -12)


def _sc_hist_body(conf_hbm, corr_hbm, out_hbm, conf_v, corr_v, pk_h, cs_h):
    wid = lax.axis_index("s") * _NC + lax.axis_index("c")
    base = wid * _CH
    pltpu.sync_copy(conf_hbm.at[pl.ds(base, _CH)], conf_v)
    pltpu.sync_copy(corr_hbm.at[pl.ds(base, _CH)], corr_v)

    zeros16 = jnp.zeros((_L,), jnp.float32)
    nrows = _NBINS - 1

    def body(i, carry):
        pks, css, cs_tot, as_tot = carry
        off = i * _L
        c = conf_v[pl.ds(off, _L)]
        a = corr_v[pl.ds(off, _L)]
        t = a * _PACK + 1.0
        new_pks = []
        new_css = []
        for j in range(1, _NBINS):
            m = c > _BOUNDS[j]
            new_pks.append(pks[j - 1] + jnp.where(m, t, 0.0))
            new_css.append(css[j - 1] + jnp.where(m, c, 0.0))
        return (tuple(new_pks), tuple(new_css), cs_tot + c, as_tot + a)

    init = (tuple(zeros16 for _ in range(nrows)),
            tuple(zeros16 for _ in range(nrows)), zeros16, zeros16)
    pks, css, cs_tot, as_tot = lax.fori_loop(0, _CH // _L, body, init)

    pk_h[0, :] = as_tot
    cs_h[0, :] = cs_tot
    for j in range(1, _NBINS):
        pk_h[j, :] = pks[j - 1]
        cs_h[j, :] = css[j - 1]
    pk_h[_NBINS, :] = zeros16
    cs_h[_NBINS, :] = zeros16

    pltpu.sync_copy(pk_h, out_hbm.at[wid, 0])
    pltpu.sync_copy(cs_h, out_hbm.at[wid, 1])


@functools.lru_cache(maxsize=1)
def _make_sc_hist():
    return pl.kernel(
        _sc_hist_body,
        mesh=plsc.VectorSubcoreMesh(core_axis_name="c", subcore_axis_name="s"),
        out_type=jax.ShapeDtypeStruct((_NW, 2, _L, _L), jnp.float32),
        scratch_types=[
            pltpu.VMEM((_CH,), jnp.float32),
            pltpu.VMEM((_CH,), jnp.float32),
            pltpu.VMEM((_L, _L), jnp.float32),
            pltpu.VMEM((_L, _L), jnp.float32),
        ],
    )


def kernel(probs, labels):
    probs4 = probs.reshape(_G2, _SB, _C, _C)
    labels4 = labels.reshape(_G2, _SB, _C)
    conf, corr = _tc_call(probs4, labels4)
    parts = _make_sc_hist()(conf.reshape(_N), corr.reshape(_N))
    pk = parts[:, 0]
    cs = parts[:, 1]
    cnt_rows = jnp.floor(pk)
    acc_rows = (pk - cnt_rows) * 4096.0
    cnt_cum = jnp.sum(cnt_rows, axis=(0, 2))
    acc_cum = jnp.sum(acc_rows, axis=(0, 2))
    cs_cum = jnp.sum(cs, axis=(0, 2))
    cnt_cum = cnt_cum.at[0].set(float(_N))
    acc_cum = acc_cum.at[0].set(jnp.sum(parts[:, 0, 0, :]))
    cs_cum = cs_cum.at[0].set(jnp.sum(parts[:, 1, 0, :]))
    counts = cnt_cum[:_NBINS] - cnt_cum[1:_NBINS + 1]
    conf_sum = cs_cum[:_NBINS] - cs_cum[1:_NBINS + 1]
    acc_sum = acc_cum[:_NBINS] - acc_cum[1:_NBINS + 1]
    denom = jnp.maximum(counts, 1.0)
    per_bin = jnp.where(counts > 0,
                        jnp.abs(conf_sum / denom - acc_sum / denom)
                        * (counts / _N), 0.0)
    return jnp.sum(per_bin).reshape(1)


# speedup vs baseline: 5.0066x; 1.1067x over previous
import functools

import numpy as np
import jax
import jax.numpy as jnp
from jax import lax
from jax.experimental import pallas as pl
from jax.experimental.pallas import tpu as pltpu
from jax.experimental.pallas import tpu_sc as plsc

_N = 1048576
_C = 128
_NBINS = 15
_R2 = 4096
_G2 = _N // _R2
_SB = _R2 // _C
_NC = 2
_NS = 16
_NW = _NC * _NS
_CH = _N // _NW
_L = 16

_BOUNDS = [float(b) for b in np.linspace(0.0, 1.0, _NBINS + 1).astype(np.float32)]


def _tc_body(probs_ref, labels_ref, conf_ref, corr_ref):
    x = probs_ref[0]
    lab = labels_ref[0]
    iota = lax.broadcasted_iota(jnp.int32, (_SB, _C, _C), 2)
    xi = lax.bitcast_convert_type(x, jnp.int32)
    key = (xi & -128) | (127 - iota)
    final = jnp.max(key, axis=2)

    pred = 127 - (final & 127)
    corr_ref[0] = jnp.where(pred == lab, 1.0, 0.0)
    conf_ref[0] = lax.bitcast_convert_type((final & -128) | 64, jnp.float32)


_tc_call = pl.pallas_call(
    _tc_body,
    grid=(_G2,),
    in_specs=[
        pl.BlockSpec((1, _SB, _C, _C), lambda i: (i, 0, 0, 0)),
        pl.BlockSpec((1, _SB, _C), lambda i: (i, 0, 0)),
    ],
    out_specs=[
        pl.BlockSpec((1, _SB, _C), lambda i: (i, 0, 0)),
        pl.BlockSpec((1, _SB, _C), lambda i: (i, 0, 0)),
    ],
    out_shape=[
        jax.ShapeDtypeStruct((_G2, _SB, _C), jnp.float32),
        jax.ShapeDtypeStruct((_G2, _SB, _C), jnp.float32),
    ],
    compiler_params=pltpu.CompilerParams(
        dimension_semantics=("arbitrary",),
    ),
)


_PACK = float(2.0 ** -12)


def _sc_hist_body(conf_hbm, corr_hbm, out_hbm, conf_v, corr_v, pk_h, cs_h):
    wid = lax.axis_index("s") * _NC + lax.axis_index("c")
    base = wid * _CH
    pltpu.sync_copy(conf_hbm.at[pl.ds(base, _CH)], conf_v)
    pltpu.sync_copy(corr_hbm.at[pl.ds(base, _CH)], corr_v)

    zeros16 = jnp.zeros((_L,), jnp.float32)
    nrows = _NBINS - 1

    def body(i, carry):
        pks, css, cs_tot, as_tot = carry
        off = i * _L
        c = conf_v[pl.ds(off, _L)]
        a = corr_v[pl.ds(off, _L)]
        t = a * _PACK + 1.0
        new_pks = []
        new_css = []
        for j in range(1, _NBINS):
            m = c > _BOUNDS[j]
            new_pks.append(pks[j - 1] + jnp.where(m, t, 0.0))
            new_css.append(css[j - 1] + jnp.where(m, c, 0.0))
        return (tuple(new_pks), tuple(new_css), cs_tot + c, as_tot + a)

    init = (tuple(zeros16 for _ in range(nrows)),
            tuple(zeros16 for _ in range(nrows)), zeros16, zeros16)
    pks, css, cs_tot, as_tot = lax.fori_loop(0, _CH // _L, body, init)

    pk_h[0, :] = as_tot
    cs_h[0, :] = cs_tot
    for j in range(1, _NBINS):
        pk_h[j, :] = pks[j - 1]
        cs_h[j, :] = css[j - 1]
    pk_h[_NBINS, :] = zeros16
    cs_h[_NBINS, :] = zeros16

    pltpu.sync_copy(pk_h, out_hbm.at[wid, 0])
    pltpu.sync_copy(cs_h, out_hbm.at[wid, 1])


@functools.lru_cache(maxsize=1)
def _make_sc_hist():
    return pl.kernel(
        _sc_hist_body,
        mesh=plsc.VectorSubcoreMesh(core_axis_name="c", subcore_axis_name="s"),
        out_type=jax.ShapeDtypeStruct((_NW, 2, _L, _L), jnp.float32),
        scratch_types=[
            pltpu.VMEM((_CH,), jnp.float32),
            pltpu.VMEM((_CH,), jnp.float32),
            pltpu.VMEM((_L, _L), jnp.float32),
            pltpu.VMEM((_L, _L), jnp.float32),
        ],
    )


def kernel(probs, labels):
    probs4 = probs.reshape(_G2, _SB, _C, _C)
    labels4 = labels.reshape(_G2, _SB, _C)
    conf, corr = _tc_call(probs4, labels4)
    parts = _make_sc_hist()(conf.reshape(_N), corr.reshape(_N))
    pk = parts[:, 0]
    cs = parts[:, 1]
    cnt_rows = jnp.floor(pk)
    acc_rows = (pk - cnt_rows) * 4096.0
    cnt_cum = jnp.sum(cnt_rows, axis=(0, 2))
    acc_cum = jnp.sum(acc_rows, axis=(0, 2))
    cs_cum = jnp.sum(cs, axis=(0, 2))
    cnt_cum = cnt_cum.at[0].set(float(_N))
    acc_cum = acc_cum.at[0].set(jnp.sum(parts[:, 0, 0, :]))
    cs_cum = cs_cum.at[0].set(jnp.sum(parts[:, 1, 0, :]))
    counts = cnt_cum[:_NBINS] - cnt_cum[1:_NBINS + 1]
    conf_sum = cs_cum[:_NBINS] - cs_cum[1:_NBINS + 1]
    acc_sum = acc_cum[:_NBINS] - acc_cum[1:_NBINS + 1]
    denom = jnp.maximum(counts, 1.0)
    per_bin = jnp.where(counts > 0,
                        jnp.abs(conf_sum / denom - acc_sum / denom)
                        * (counts / _N), 0.0)
    return jnp.sum(per_bin).reshape(1)


# speedup vs baseline: 5.1737x; 1.0334x over previous
import functools

import numpy as np
import jax
import jax.numpy as jnp
from jax import lax
from jax.experimental import pallas as pl
from jax.experimental.pallas import tpu as pltpu
from jax.experimental.pallas import tpu_sc as plsc

_N = 1048576
_C = 128
_NBINS = 15
_R2 = 8192
_G2 = _N // _R2
_SB = _R2 // _C
_NC = 2
_NS = 16
_NW = _NC * _NS
_CH = _N // _NW
_L = 16

_BOUNDS = [float(b) for b in np.linspace(0.0, 1.0, _NBINS + 1).astype(np.float32)]


def _tc_body(probs_ref, labels_ref, conf_ref, corr_ref):
    x = probs_ref[0]
    lab = labels_ref[0]
    iota = lax.broadcasted_iota(jnp.int32, (_SB, _C, _C), 2)
    xi = lax.bitcast_convert_type(x, jnp.int32)
    key = (xi & -128) | (127 - iota)
    final = jnp.max(key, axis=2)

    pred = 127 - (final & 127)
    corr_ref[0] = jnp.where(pred == lab, 1.0, 0.0)
    conf_ref[0] = lax.bitcast_convert_type((final & -128) | 64, jnp.float32)


_tc_call = pl.pallas_call(
    _tc_body,
    grid=(_G2,),
    in_specs=[
        pl.BlockSpec((1, _SB, _C, _C), lambda i: (i, 0, 0, 0)),
        pl.BlockSpec((1, _SB, _C), lambda i: (i, 0, 0)),
    ],
    out_specs=[
        pl.BlockSpec((1, _SB, _C), lambda i: (i, 0, 0)),
        pl.BlockSpec((1, _SB, _C), lambda i: (i, 0, 0)),
    ],
    out_shape=[
        jax.ShapeDtypeStruct((_G2, _SB, _C), jnp.float32),
        jax.ShapeDtypeStruct((_G2, _SB, _C), jnp.float32),
    ],
    compiler_params=pltpu.CompilerParams(
        dimension_semantics=("arbitrary",),
    ),
)


_PACK = float(2.0 ** -12)


def _sc_hist_body(conf_hbm, corr_hbm, out_hbm, conf_v, corr_v, pk_h, cs_h):
    wid = lax.axis_index("s") * _NC + lax.axis_index("c")
    base = wid * _CH
    pltpu.sync_copy(conf_hbm.at[pl.ds(base, _CH)], conf_v)
    pltpu.sync_copy(corr_hbm.at[pl.ds(base, _CH)], corr_v)

    zeros16 = jnp.zeros((_L,), jnp.float32)
    nrows = _NBINS - 1

    def body(i, carry):
        pks, css, cs_tot, as_tot = carry
        off = i * _L
        c = conf_v[pl.ds(off, _L)]
        a = corr_v[pl.ds(off, _L)]
        t = a * _PACK + 1.0
        new_pks = []
        new_css = []
        for j in range(1, _NBINS):
            m = c > _BOUNDS[j]
            new_pks.append(pks[j - 1] + jnp.where(m, t, 0.0))
            new_css.append(css[j - 1] + jnp.where(m, c, 0.0))
        return (tuple(new_pks), tuple(new_css), cs_tot + c, as_tot + a)

    init = (tuple(zeros16 for _ in range(nrows)),
            tuple(zeros16 for _ in range(nrows)), zeros16, zeros16)
    pks, css, cs_tot, as_tot = lax.fori_loop(0, _CH // _L, body, init)

    pk_h[0, :] = as_tot
    cs_h[0, :] = cs_tot
    for j in range(1, _NBINS):
        pk_h[j, :] = pks[j - 1]
        cs_h[j, :] = css[j - 1]
    pk_h[_NBINS, :] = zeros16
    cs_h[_NBINS, :] = zeros16

    pltpu.sync_copy(pk_h, out_hbm.at[wid, 0])
    pltpu.sync_copy(cs_h, out_hbm.at[wid, 1])


@functools.lru_cache(maxsize=1)
def _make_sc_hist():
    return pl.kernel(
        _sc_hist_body,
        mesh=plsc.VectorSubcoreMesh(core_axis_name="c", subcore_axis_name="s"),
        out_type=jax.ShapeDtypeStruct((_NW, 2, _L, _L), jnp.float32),
        scratch_types=[
            pltpu.VMEM((_CH,), jnp.float32),
            pltpu.VMEM((_CH,), jnp.float32),
            pltpu.VMEM((_L, _L), jnp.float32),
            pltpu.VMEM((_L, _L), jnp.float32),
        ],
    )


def kernel(probs, labels):
    probs4 = probs.reshape(_G2, _SB, _C, _C)
    labels4 = labels.reshape(_G2, _SB, _C)
    conf, corr = _tc_call(probs4, labels4)
    parts = _make_sc_hist()(conf.reshape(_N), corr.reshape(_N))
    pk = parts[:, 0]
    cs = parts[:, 1]
    cnt_rows = jnp.floor(pk)
    acc_rows = (pk - cnt_rows) * 4096.0
    cnt_cum = jnp.sum(cnt_rows, axis=(0, 2))
    acc_cum = jnp.sum(acc_rows, axis=(0, 2))
    cs_cum = jnp.sum(cs, axis=(0, 2))
    cnt_cum = cnt_cum.at[0].set(float(_N))
    acc_cum = acc_cum.at[0].set(jnp.sum(parts[:, 0, 0, :]))
    cs_cum = cs_cum.at[0].set(jnp.sum(parts[:, 1, 0, :]))
    counts = cnt_cum[:_NBINS] - cnt_cum[1:_NBINS + 1]
    conf_sum = cs_cum[:_NBINS] - cs_cum[1:_NBINS + 1]
    acc_sum = acc_cum[:_NBINS] - acc_cum[1:_NBINS + 1]
    denom = jnp.maximum(counts, 1.0)
    per_bin = jnp.where(counts > 0,
                        jnp.abs(conf_sum / denom - acc_sum / denom)
                        * (counts / _N), 0.0)
    return jnp.sum(per_bin).reshape(1)


# speedup vs baseline: 5.2558x; 1.0159x over previous
import functools

import numpy as np
import jax
import jax.numpy as jnp
from jax import lax
from jax.experimental import pallas as pl
from jax.experimental.pallas import tpu as pltpu
from jax.experimental.pallas import tpu_sc as plsc

_N = 1048576
_C = 128
_NBINS = 15
_R2 = 16384
_G2 = _N // _R2
_SB = _R2 // _C
_NC = 2
_NS = 16
_NW = _NC * _NS
_CH = _N // _NW
_L = 16

_BOUNDS = [float(b) for b in np.linspace(0.0, 1.0, _NBINS + 1).astype(np.float32)]


def _tc_body(probs_ref, labels_ref, conf_ref, corr_ref):
    x = probs_ref[0]
    lab = labels_ref[0]
    iota = lax.broadcasted_iota(jnp.int32, (_SB, _C, _C), 2)
    xi = lax.bitcast_convert_type(x, jnp.int32)
    key = (xi & -128) | (127 - iota)
    final = jnp.max(key, axis=2)

    pred = 127 - (final & 127)
    corr_ref[0] = jnp.where(pred == lab, 1.0, 0.0)
    conf_ref[0] = lax.bitcast_convert_type((final & -128) | 64, jnp.float32)


_tc_call = pl.pallas_call(
    _tc_body,
    grid=(_G2,),
    in_specs=[
        pl.BlockSpec((1, _SB, _C, _C), lambda i: (i, 0, 0, 0)),
        pl.BlockSpec((1, _SB, _C), lambda i: (i, 0, 0)),
    ],
    out_specs=[
        pl.BlockSpec((1, _SB, _C), lambda i: (i, 0, 0)),
        pl.BlockSpec((1, _SB, _C), lambda i: (i, 0, 0)),
    ],
    out_shape=[
        jax.ShapeDtypeStruct((_G2, _SB, _C), jnp.float32),
        jax.ShapeDtypeStruct((_G2, _SB, _C), jnp.float32),
    ],
    compiler_params=pltpu.CompilerParams(
        dimension_semantics=("arbitrary",),
    ),
)


_PACK = float(2.0 ** -12)


def _sc_hist_body(conf_hbm, corr_hbm, out_hbm, conf_v, corr_v, pk_h, cs_h):
    wid = lax.axis_index("s") * _NC + lax.axis_index("c")
    base = wid * _CH
    pltpu.sync_copy(conf_hbm.at[pl.ds(base, _CH)], conf_v)
    pltpu.sync_copy(corr_hbm.at[pl.ds(base, _CH)], corr_v)

    zeros16 = jnp.zeros((_L,), jnp.float32)
    nrows = _NBINS - 1

    def body(i, carry):
        pks, css, cs_tot, as_tot = carry
        off = i * _L
        c = conf_v[pl.ds(off, _L)]
        a = corr_v[pl.ds(off, _L)]
        t = a * _PACK + 1.0
        new_pks = []
        new_css = []
        for j in range(1, _NBINS):
            m = c > _BOUNDS[j]
            new_pks.append(pks[j - 1] + jnp.where(m, t, 0.0))
            new_css.append(css[j - 1] + jnp.where(m, c, 0.0))
        return (tuple(new_pks), tuple(new_css), cs_tot + c, as_tot + a)

    init = (tuple(zeros16 for _ in range(nrows)),
            tuple(zeros16 for _ in range(nrows)), zeros16, zeros16)
    pks, css, cs_tot, as_tot = lax.fori_loop(0, _CH // _L, body, init)

    pk_h[0, :] = as_tot
    cs_h[0, :] = cs_tot
    for j in range(1, _NBINS):
        pk_h[j, :] = pks[j - 1]
        cs_h[j, :] = css[j - 1]
    pk_h[_NBINS, :] = zeros16
    cs_h[_NBINS, :] = zeros16

    pltpu.sync_copy(pk_h, out_hbm.at[wid, 0])
    pltpu.sync_copy(cs_h, out_hbm.at[wid, 1])


@functools.lru_cache(maxsize=1)
def _make_sc_hist():
    return pl.kernel(
        _sc_hist_body,
        mesh=plsc.VectorSubcoreMesh(core_axis_name="c", subcore_axis_name="s"),
        out_type=jax.ShapeDtypeStruct((_NW, 2, _L, _L), jnp.float32),
        scratch_types=[
            pltpu.VMEM((_CH,), jnp.float32),
            pltpu.VMEM((_CH,), jnp.float32),
            pltpu.VMEM((_L, _L), jnp.float32),
            pltpu.VMEM((_L, _L), jnp.float32),
        ],
    )


def kernel(probs, labels):
    probs4 = probs.reshape(_G2, _SB, _C, _C)
    labels4 = labels.reshape(_G2, _SB, _C)
    conf, corr = _tc_call(probs4, labels4)
    parts = _make_sc_hist()(conf.reshape(_N), corr.reshape(_N))
    pk = parts[:, 0]
    cs = parts[:, 1]
    cnt_rows = jnp.floor(pk)
    acc_rows = (pk - cnt_rows) * 4096.0
    cnt_cum = jnp.sum(cnt_rows, axis=(0, 2))
    acc_cum = jnp.sum(acc_rows, axis=(0, 2))
    cs_cum = jnp.sum(cs, axis=(0, 2))
    cnt_cum = cnt_cum.at[0].set(float(_N))
    acc_cum = acc_cum.at[0].set(jnp.sum(parts[:, 0, 0, :]))
    cs_cum = cs_cum.at[0].set(jnp.sum(parts[:, 1, 0, :]))
    counts = cnt_cum[:_NBINS] - cnt_cum[1:_NBINS + 1]
    conf_sum = cs_cum[:_NBINS] - cs_cum[1:_NBINS + 1]
    acc_sum = acc_cum[:_NBINS] - acc_cum[1:_NBINS + 1]
    denom = jnp.maximum(counts, 1.0)
    per_bin = jnp.where(counts > 0,
                        jnp.abs(conf_sum / denom - acc_sum / denom)
                        * (counts / _N), 0.0)
    return jnp.sum(per_bin).reshape(1)
